# baseline (speedup 1.0000x reference)
import functools

import jax
import jax.numpy as jnp
from jax import lax
from jax.experimental import pallas as pl
from jax.experimental.pallas import tpu as pltpu
from jax.experimental.pallas import tpu_sc as plsc

_N = 10000
_E = 160000
_WN = 32
_WK = 64
_DEPTH = 4

_NP = 10016
_NPS = _NP // 16
_EP = 245760
_G = _EP // 8
_NW = 32
_RPT = _EP // _NW
_CH = 128
_NCH = _RPT // _CH
_GPT = _G // _NW
_SCH = 16
_NSCH = _GPT // _SCH

_MB = 1024
_MG = _MB // 8

_RING = 8


def _sc_gather_body(h_hbm, idx_hbm, out_hbm, idxb, bufs, gsem, ssem):
    wid = lax.axis_index("s") * 2 + lax.axis_index("c")
    base = wid * _RPT
    pltpu.sync_copy(idx_hbm.at[wid], idxb)
    gds = [None] * _NCH
    sds = [None] * _NCH
    for j in range(_NCH):
        b = j % _RING
        if j >= _RING:
            sds[j - _RING].wait()
        gds[j] = pltpu.async_copy(h_hbm.at[idxb.at[j]], bufs.at[b], gsem.at[b])
        if j >= _RING - 1:
            k = j - _RING + 1
            gds[k].wait()
            sds[k] = pltpu.async_copy(
                bufs.at[k % _RING], out_hbm.at[pl.ds(base + k * _CH, _CH)],
                ssem.at[k % _RING])
    for k in range(_NCH - _RING + 1, _NCH):
        gds[k].wait()
        sds[k] = pltpu.async_copy(
            bufs.at[k % _RING], out_hbm.at[pl.ds(base + k * _CH, _CH)],
            ssem.at[k % _RING])
    for k in range(_NCH - _RING, _NCH):
        sds[k].wait()


def _sc_scatter_body(aggp_hbm, gnode_hbm, zeros_hbm, out_hbm, rows_v, idx_v,
                     stripe_v, shared, sem):
    cid = lax.axis_index("c")
    sid = lax.axis_index("s")
    wid = sid * 2 + cid
    gbase = wid * _GPT
    pltpu.sync_copy(aggp_hbm.at[pl.ds(gbase, _GPT)], rows_v)
    pltpu.sync_copy(gnode_hbm.at[pl.ds(gbase, _GPT)], idx_v)
    sbase = sid * _NPS
    pltpu.sync_copy(zeros_hbm.at[pl.ds(sbase, _NPS)], stripe_v)
    pltpu.sync_copy(stripe_v, shared.at[pl.ds(sbase, _NPS)])
    plsc.subcore_barrier()
    ds = []
    for j in range(_NSCH):
        idxv = idx_v[pl.ds(j * _SCH, _SCH)]
        ds.append(pltpu.async_copy(
            rows_v.at[pl.ds(j * _SCH, _SCH)], shared.at[idxv], sem, add=True))
    for d in ds:
        d.wait()
    plsc.subcore_barrier()
    pltpu.sync_copy(shared.at[pl.ds(sbase, _NPS)], stripe_v)
    pltpu.sync_copy(stripe_v, out_hbm.at[cid].at[pl.ds(sbase, _NPS)])


@functools.cache
def _sc_kernels():
    mesh = plsc.VectorSubcoreMesh(core_axis_name="c", subcore_axis_name="s",
                                  num_cores=2, num_subcores=16)
    params = pltpu.CompilerParams(use_tc_tiling_on_sc=False)
    gather = pl.kernel(
        _sc_gather_body,
        out_type=jax.ShapeDtypeStruct((_EP, _WN), jnp.float32),
        mesh=mesh,
        compiler_params=params,
        scratch_types=[
            pltpu.VMEM((_NCH, _CH), jnp.int32),
            pltpu.VMEM((_RING, _CH, _WN), jnp.float32),
            pltpu.SemaphoreType.DMA((_RING,)),
            pltpu.SemaphoreType.DMA((_RING,)),
        ],
        name="scgather",
    )
    scatter = pl.kernel(
        _sc_scatter_body,
        out_type=jax.ShapeDtypeStruct((2, _NP, _WN), jnp.float32),
        mesh=mesh,
        compiler_params=params,
        scratch_types=[
            pltpu.VMEM((_GPT, _WN), jnp.float32),
            pltpu.VMEM((_GPT,), jnp.int32),
            pltpu.VMEM((_NPS, _WN), jnp.float32),
            pltpu.VMEM_SHARED((_NP, _WN), jnp.float32),
            pltpu.SemaphoreType.DMA,
        ],
        name="scscatter",
    )
    return gather, scatter


def _sc_gather(h, idx3):
    return _sc_kernels()[0](h, idx3)


def _sc_scatter(aggp, gnode, zeros_np):
    return _sc_kernels()[1](aggp, gnode, zeros_np)


def _mlp_body(ea_ref, k1w_ref, k1b_ref, k2w_ref, k2b_ref, out_ref):
    e1 = jnp.maximum(
        jnp.dot(ea_ref[...], k1w_ref[...], preferred_element_type=jnp.float32)
        + k1b_ref[...], 0.0)
    out_ref[...] = jnp.maximum(
        jnp.dot(e1, k2w_ref[...], preferred_element_type=jnp.float32)
        + k2b_ref[...], 0.0)


def _main_body(hs_ref, cs_ref, k3r_ref, b3r_ref, out_ref):
    hs = hs_ref[...]
    cs = cs_ref[...]
    csg = cs.reshape(_MG, 8, _WK)
    hsg = hs.reshape(_MG, 8, _WN)
    m = lax.dot_general(csg, hsg, (((1,), (1,)), ((0,), (0,))),
                        preferred_element_type=jnp.float32)
    hsum = jnp.sum(hsg, axis=1)
    out_ref[...] = (
        jnp.dot(m.reshape(_MG, _WK * _WN), k3r_ref[...],
                preferred_element_type=jnp.float32)
        + jnp.dot(hsum, b3r_ref[...], preferred_element_type=jnp.float32))


def _update_body(agg_ref, h_ref, deginv_ref, root_ref, bias_ref, out_ref,
                 *, relu):
    agg = agg_ref[0] + agg_ref[1]
    hn = (agg * deginv_ref[...]
          + jnp.dot(h_ref[...], root_ref[...],
                    preferred_element_type=jnp.float32)
          + bias_ref[...])
    if relu:
        hn = jnp.maximum(hn, 0.0)
    rows = lax.broadcasted_iota(jnp.int32, (_NP, _WN), 0) < _N
    out_ref[...] = jnp.where(rows, hn, 0.0)


def _final_body(h_ref, w2_ref, b2_ref, w3t_ref, b3_ref, out_ref):
    h2 = jnp.maximum(
        jnp.dot(h_ref[...], w2_ref[...], preferred_element_type=jnp.float32)
        + b2_ref[...], 0.0)
    out_ref[...] = (jnp.sum(h2 * w3t_ref[...], axis=1, keepdims=True)
                    + b3_ref[...])


def _mlp(ea_pad, k1_w, k1_b, k2_w, k2_b):
    blk = 2048
    return pl.pallas_call(
        _mlp_body,
        grid=(_EP // blk,),
        in_specs=[
            pl.BlockSpec((blk, 4), lambda i: (i, 0)),
            pl.BlockSpec((4, _WK // 2), lambda i: (0, 0)),
            pl.BlockSpec((_WK // 2,), lambda i: (0,)),
            pl.BlockSpec((_WK // 2, _WK), lambda i: (0, 0)),
            pl.BlockSpec((_WK,), lambda i: (0,)),
        ],
        out_specs=pl.BlockSpec((blk, _WK), lambda i: (i, 0)),
        out_shape=jax.ShapeDtypeStruct((_EP, _WK), jnp.float32),
        name="tcmlp",
    )(ea_pad, k1_w, k1_b, k2_w, k2_b)


def _main(hs, cs, k3r, b3r):
    return pl.pallas_call(
        _main_body,
        grid=(_EP // _MB,),
        in_specs=[
            pl.BlockSpec((_MB, _WN), lambda i: (i, 0)),
            pl.BlockSpec((_MB, _WK), lambda i: (i, 0)),
            pl.BlockSpec((_WK * _WN, _WN), lambda i: (0, 0)),
            pl.BlockSpec((_WN, _WN), lambda i: (0, 0)),
        ],
        out_specs=pl.BlockSpec((_MG, _WN), lambda i: (i, 0)),
        out_shape=jax.ShapeDtypeStruct((_G, _WN), jnp.float32),
        name="tcmain",
    )(hs, cs, k3r, b3r)


def _update(agg2, h, deginv, root, bias, relu):
    return pl.pallas_call(
        functools.partial(_update_body, relu=relu),
        in_specs=[
            pl.BlockSpec((2, _NP, _WN), lambda: (0, 0, 0)),
            pl.BlockSpec((_NP, _WN), lambda: (0, 0)),
            pl.BlockSpec((_NP, 1), lambda: (0, 0)),
            pl.BlockSpec((_WN, _WN), lambda: (0, 0)),
            pl.BlockSpec((_WN,), lambda: (0,)),
        ],
        out_specs=pl.BlockSpec((_NP, _WN), lambda: (0, 0)),
        out_shape=jax.ShapeDtypeStruct((_NP, _WN), jnp.float32),
        name="tcupdate",
    )(agg2, h, deginv, root, bias)


def _final(h, fc2_w, fc2_b, fc3_w, fc3_b):
    return pl.pallas_call(
        _final_body,
        in_specs=[
            pl.BlockSpec((_NP, _WN), lambda: (0, 0)),
            pl.BlockSpec((_WN, 128), lambda: (0, 0)),
            pl.BlockSpec((128,), lambda: (0,)),
            pl.BlockSpec((1, 128), lambda: (0, 0)),
            pl.BlockSpec((1,), lambda: (0,)),
        ],
        out_specs=pl.BlockSpec((_NP, 1), lambda: (0, 0)),
        out_shape=jax.ShapeDtypeStruct((_NP, 1), jnp.float32),
        name="tcfinal",
    )(h, fc2_w, fc2_b, fc3_w.T, fc3_b)


def kernel(x, edge_index, edge_attr, fc1_w, fc1_b, k1_w, k1_b, k2_w, k2_b,
           k3_w, k3_b, root, conv_bias, fc2_w, fc2_b, fc3_w, fc3_b):
    src = edge_index[0]
    dst = edge_index[1]

    key = (dst.astype(jnp.uint32) << 18) | jnp.arange(_E, dtype=jnp.uint32)
    key_s = jnp.sort(key)
    order = (key_s & jnp.uint32((1 << 18) - 1)).astype(jnp.int32)
    dst_s = (key_s >> 18).astype(jnp.int32)
    src_s = jnp.take(src, order)
    starts = jnp.searchsorted(
        dst_s, jnp.arange(_N + 1, dtype=jnp.int32)).astype(jnp.int32)
    deg = starts[1:] - starts[:-1]
    gcnt = (deg + 7) // 8
    goff = jnp.concatenate(
        [jnp.zeros((1,), jnp.int32), jnp.cumsum(gcnt).astype(jnp.int32)])
    total_g = goff[_N]
    garange = jnp.arange(_G, dtype=jnp.int32)
    gnode_raw = (jnp.searchsorted(goff, garange, side='right') - 1).astype(jnp.int32)
    gnode_c = jnp.minimum(gnode_raw, _N)
    deg_ext = jnp.concatenate([deg, jnp.zeros((1,), jnp.int32)])
    starts_ext = jnp.concatenate([starts[:_N], jnp.full((1,), _E, jnp.int32)])
    sidx = jnp.arange(_EP, dtype=jnp.int32)
    gs = sidx // 8
    node_s = jnp.take(gnode_c, gs)
    rank = sidx - 8 * jnp.take(goff, node_s)
    valid = rank < jnp.take(deg_ext, node_s)
    epos = jnp.clip(jnp.take(starts_ext, node_s) + rank, 0, _E - 1)
    idx = jnp.where(valid, jnp.take(src_s, epos),
                    _N + (sidx % 16)).astype(jnp.int32)
    eid = jnp.where(valid, jnp.take(order, epos), 0).astype(jnp.int32)
    gnode = jnp.where(garange < total_g, gnode_c,
                      _N + (garange % 16)).astype(jnp.int32)
    deginv = 1.0 / jnp.clip(deg.astype(jnp.float32), 1.0)
    deginv_ext = jnp.concatenate(
        [deginv, jnp.ones((_NP - _N,), jnp.float32)])[:, None]
    idx3 = idx.reshape(_NW, _NCH, _CH)

    ea_pad = jnp.take(edge_attr, eid, axis=0)
    cs = _mlp(ea_pad, k1_w, k1_b, k2_w, k2_b)
    k3r = k3_w.reshape(_WK * _WN, _WN)
    b3r = k3_b.reshape(_WN, _WN)

    h0 = x @ fc1_w + fc1_b
    h = jnp.zeros((_NP, _WN), jnp.float32).at[:_N].set(h0)
    zeros_np = jnp.zeros((_NP, _WN), jnp.float32)

    for d in range(_DEPTH):
        hs = _sc_gather(h, idx3)
        aggp = _main(hs, cs, k3r, b3r)
        agg2 = _sc_scatter(aggp, gnode, zeros_np)
        h = _update(agg2, h, deginv_ext, root, conv_bias, d != _DEPTH - 1)

    out = _final(h, fc2_w, fc2_b, fc3_w, fc3_b)
    return out[:_N]


# speedup vs baseline: 2.7922x; 2.7922x over previous
import functools

import jax
import jax.numpy as jnp
from jax import lax
from jax.experimental import pallas as pl
from jax.experimental.pallas import tpu as pltpu
from jax.experimental.pallas import tpu_sc as plsc

_N = 10000
_E = 160000
_WN = 32
_WK = 64
_DEPTH = 4

_NP = 10016
_NPS = _NP // 16
_EP = 245760
_G = _EP // 8
_NW = 32
_RPT = _EP // _NW
_CH = 128
_NCH = _RPT // _CH
_GPT = _G // _NW
_SCH = 16
_NSCH = _GPT // _SCH
_KMASK = (1 << 18) - 1

_MB = 1024
_MG = _MB // 8

_PREC = lax.Precision.HIGHEST


def _iota16():
    return lax.broadcasted_iota(jnp.int32, (16,), 0)


def _sc_build_body(keys_hbm, src_hbm, cs_hbm, goff_hbm, starts_hbm, deg_hbm,
                   idx_hbm, csp_hbm, gnode_hbm,
                   gofft, startst, degt,
                   eposb, validb, nodeb, gnb, kb, eob, csb, srcb, idxob,
                   k1sem, g2sem, ssem):
    wid = lax.axis_index("s") * 2 + lax.axis_index("c")
    sbase = wid * _RPT
    gbase = wid * _GPT
    pltpu.sync_copy(goff_hbm, gofft)
    pltpu.sync_copy(starts_hbm, startst)
    pltpu.sync_copy(deg_hbm, degt)

    def phase_b(c, p):
        for v in range(8):
            sv = sbase + c * _CH + v * 16 + _iota16()
            gvec = sv >> 3
            lo = jnp.zeros((16,), jnp.int32)
            hi = jnp.full((16,), _N, jnp.int32)
            for _ in range(14):
                mid = (lo + hi + 1) >> 1
                gm = plsc.load_gather(gofft, [mid])
                cond = gm <= gvec
                lo = jnp.where(cond, mid, lo)
                hi = jnp.where(cond, hi, mid - 1)
            node = lo
            go = plsc.load_gather(gofft, [node])
            st = plsc.load_gather(startst, [node])
            dg = plsc.load_gather(degt, [node])
            rank = sv - 8 * go
            valid = rank < dg
            epos = jnp.minimum(jnp.maximum(st + rank, 0), _E - 1)
            eposb[p, pl.ds(v * 16, 16)] = epos
            validb[p, pl.ds(v * 16, 16)] = jnp.where(valid, 1, 0)
            nodeb[p, pl.ds(v * 16, 16)] = node
        pltpu.async_copy(keys_hbm.at[eposb.at[p]], kb.at[p], k1sem)

    def phase_c(c, p):
        pltpu.make_async_copy(keys_hbm.at[pl.ds(0, _CH)], kb.at[p],
                              k1sem).wait()
        for v in range(8):
            kv = kb[p, pl.ds(v * 16, 16)]
            eob[p, pl.ds(v * 16, 16)] = kv & _KMASK
        pltpu.async_copy(src_hbm.at[eob.at[p]], srcb.at[p], g2sem)
        pltpu.async_copy(cs_hbm.at[eob.at[p]], csb.at[p], g2sem)

    def phase_d(c, p, first):
        pltpu.make_async_copy(src_hbm.at[pl.ds(0, _CH)], srcb.at[p],
                              g2sem).wait()
        pltpu.make_async_copy(cs_hbm.at[pl.ds(0, _CH)], csb.at[p],
                              g2sem).wait()

        def wait_stores():
            pltpu.make_async_copy(idxob.at[p],
                                  idx_hbm.at[pl.ds(0, _CH)], ssem).wait()
            pltpu.make_async_copy(csb.at[p],
                                  csp_hbm.at[pl.ds(0, _CH)], ssem).wait()
            pltpu.make_async_copy(gnb.at[p],
                                  gnode_hbm.at[pl.ds(0, 16)], ssem).wait()

        pl.when(jnp.logical_not(first))(wait_stores)
        for v in range(8):
            sv = sbase + c * _CH + v * 16 + _iota16()
            va = validb[p, pl.ds(v * 16, 16)]
            sr = srcb[p, pl.ds(v * 16, 16)]
            idxob[p, pl.ds(v * 16, 16)] = jnp.where(
                va != 0, sr, _N + (sv & 15))
        n16 = plsc.load_gather(nodeb.at[p], [_iota16() * 8])
        g16 = gbase + c * 16 + _iota16()
        gnb[p, pl.ds(0, 16)] = jnp.where(n16 < _N, n16, _N + (g16 & 15))
        pltpu.async_copy(idxob.at[p],
                         idx_hbm.at[pl.ds(sbase + c * _CH, _CH)], ssem)
        pltpu.async_copy(csb.at[p],
                         csp_hbm.at[pl.ds(sbase + c * _CH, _CH)], ssem)
        pltpu.async_copy(gnb.at[p],
                         gnode_hbm.at[pl.ds(gbase + c * 16, 16)], ssem)

    def step(j, p):
        pl.when(j >= 2)(lambda: phase_d(j - 2, p, j <= 2))
        pl.when(jnp.logical_and(j >= 1, j <= _NCH))(
            lambda: phase_c(j - 1, 1 - p))
        pl.when(j <= _NCH - 1)(lambda: phase_b(j, p))

    def body(t, carry):
        step(2 * t, 0)
        step(2 * t + 1, 1)
        return carry

    lax.fori_loop(0, (_NCH + 2) // 2, body, 0)
    pltpu.make_async_copy(idxob.at[1], idx_hbm.at[pl.ds(0, _CH)], ssem).wait()
    pltpu.make_async_copy(csb.at[1], csp_hbm.at[pl.ds(0, _CH)], ssem).wait()
    pltpu.make_async_copy(gnb.at[1], gnode_hbm.at[pl.ds(0, 16)], ssem).wait()


_RING = 8


def _sc_gather_body(h_hbm, idx_hbm, out_hbm, idxb, bufs, stripe_v, shared,
                    gsem, ssem):
    cid = lax.axis_index("c")
    sid = lax.axis_index("s")
    wid = sid * 2 + cid
    base = wid * _RPT
    hrow = sid * _NPS
    pltpu.sync_copy(h_hbm.at[pl.ds(hrow, _NPS)], stripe_v)
    pltpu.sync_copy(stripe_v, shared.at[pl.ds(hrow, _NPS)])
    pltpu.sync_copy(idx_hbm.at[wid], idxb)
    plsc.subcore_barrier()
    gds = [None] * _NCH
    sds = [None] * _NCH
    for j in range(_NCH):
        b = j % _RING
        if j >= _RING:
            sds[j - _RING].wait()
        gds[j] = pltpu.async_copy(shared.at[idxb.at[j]], bufs.at[b],
                                  gsem.at[b])
        if j >= _RING - 1:
            k = j - _RING + 1
            gds[k].wait()
            sds[k] = pltpu.async_copy(
                bufs.at[k % _RING], out_hbm.at[pl.ds(base + k * _CH, _CH)],
                ssem.at[k % _RING])
    for k in range(_NCH - _RING + 1, _NCH):
        gds[k].wait()
        sds[k] = pltpu.async_copy(
            bufs.at[k % _RING], out_hbm.at[pl.ds(base + k * _CH, _CH)],
            ssem.at[k % _RING])
    for k in range(_NCH - _RING, _NCH):
        sds[k].wait()


def _sc_scatter_body(aggp_hbm, gnode_hbm, zeros_hbm, out_hbm, rows_v, idx_v,
                     stripe_v, shared, sem):
    cid = lax.axis_index("c")
    sid = lax.axis_index("s")
    wid = sid * 2 + cid
    gbase = wid * _GPT
    pltpu.sync_copy(aggp_hbm.at[pl.ds(gbase, _GPT)], rows_v)
    pltpu.sync_copy(gnode_hbm.at[pl.ds(gbase, _GPT)], idx_v)
    sbase = sid * _NPS
    pltpu.sync_copy(zeros_hbm.at[pl.ds(sbase, _NPS)], stripe_v)
    pltpu.sync_copy(stripe_v, shared.at[pl.ds(sbase, _NPS)])
    plsc.subcore_barrier()
    ds = []
    for j in range(_NSCH):
        idxv = idx_v[pl.ds(j * _SCH, _SCH)]
        ds.append(pltpu.async_copy(
            rows_v.at[pl.ds(j * _SCH, _SCH)], shared.at[idxv], sem, add=True))
    for d in ds:
        d.wait()
    plsc.subcore_barrier()
    pltpu.sync_copy(shared.at[pl.ds(sbase, _NPS)], stripe_v)
    pltpu.sync_copy(stripe_v, out_hbm.at[cid].at[pl.ds(sbase, _NPS)])


@functools.cache
def _sc_kernels():
    mesh = plsc.VectorSubcoreMesh(core_axis_name="c", subcore_axis_name="s",
                                  num_cores=2, num_subcores=16)
    params = pltpu.CompilerParams(use_tc_tiling_on_sc=False)
    build_params = pltpu.CompilerParams(use_tc_tiling_on_sc=False,
                                        needs_layout_passes=False)
    build = pl.kernel(
        _sc_build_body,
        out_type=(jax.ShapeDtypeStruct((_EP,), jnp.int32),
                  jax.ShapeDtypeStruct((_EP, _WK), jnp.float32),
                  jax.ShapeDtypeStruct((_G,), jnp.int32)),
        mesh=mesh,
        compiler_params=build_params,
        scratch_types=[
            pltpu.VMEM((_NP,), jnp.int32),
            pltpu.VMEM((_NP,), jnp.int32),
            pltpu.VMEM((_NP,), jnp.int32),
            pltpu.VMEM((2, _CH), jnp.int32),
            pltpu.VMEM((2, _CH), jnp.int32),
            pltpu.VMEM((2, _CH), jnp.int32),
            pltpu.VMEM((2, 16), jnp.int32),
            pltpu.VMEM((2, _CH), jnp.int32),
            pltpu.VMEM((2, _CH), jnp.int32),
            pltpu.VMEM((2, _CH, _WK), jnp.float32),
            pltpu.VMEM((2, _CH), jnp.int32),
            pltpu.VMEM((2, _CH), jnp.int32),
            pltpu.SemaphoreType.DMA,
            pltpu.SemaphoreType.DMA,
            pltpu.SemaphoreType.DMA,
        ],
        name="scbuild",
    )
    gather = pl.kernel(
        _sc_gather_body,
        out_type=jax.ShapeDtypeStruct((_EP, _WN), jnp.float32),
        mesh=mesh,
        compiler_params=params,
        scratch_types=[
            pltpu.VMEM((_NCH, _CH), jnp.int32),
            pltpu.VMEM((_RING, _CH, _WN), jnp.float32),
            pltpu.VMEM((_NPS, _WN), jnp.float32),
            pltpu.VMEM_SHARED((_NP, _WN), jnp.float32),
            pltpu.SemaphoreType.DMA((_RING,)),
            pltpu.SemaphoreType.DMA((_RING,)),
        ],
        name="scgather",
    )
    scatter = pl.kernel(
        _sc_scatter_body,
        out_type=jax.ShapeDtypeStruct((2, _NP, _WN), jnp.float32),
        mesh=mesh,
        compiler_params=params,
        scratch_types=[
            pltpu.VMEM((_GPT, _WN), jnp.float32),
            pltpu.VMEM((_GPT,), jnp.int32),
            pltpu.VMEM((_NPS, _WN), jnp.float32),
            pltpu.VMEM_SHARED((_NP, _WN), jnp.float32),
            pltpu.SemaphoreType.DMA,
        ],
        name="scscatter",
    )
    return build, gather, scatter


def _mlp_body(ea_ref, k1w_ref, k1b_ref, k2w_ref, k2b_ref, out_ref):
    e1 = jnp.maximum(
        jnp.dot(ea_ref[...], k1w_ref[...], preferred_element_type=jnp.float32,
                precision=_PREC) + k1b_ref[...], 0.0)
    out_ref[...] = jnp.maximum(
        jnp.dot(e1, k2w_ref[...], preferred_element_type=jnp.float32,
                precision=_PREC) + k2b_ref[...], 0.0)


def _main_body(hs_ref, cs_ref, k3r_ref, b3r_ref, out_ref):
    hs = hs_ref[...]
    cs = cs_ref[...]
    csg = cs.reshape(_MG, 8, _WK)
    hsg = hs.reshape(_MG, 8, _WN)
    m = lax.dot_general(csg, hsg, (((1,), (1,)), ((0,), (0,))),
                        preferred_element_type=jnp.float32,
                        precision=_PREC)
    hsum = jnp.sum(hsg, axis=1)
    out_ref[...] = (
        jnp.dot(m.reshape(_MG, _WK * _WN), k3r_ref[...],
                preferred_element_type=jnp.float32, precision=_PREC)
        + jnp.dot(hsum, b3r_ref[...], preferred_element_type=jnp.float32,
                  precision=_PREC))


def _update_body(agg_ref, h_ref, deginv_ref, root_ref, bias_ref, out_ref,
                 *, relu):
    agg = agg_ref[0] + agg_ref[1]
    hn = (agg * deginv_ref[...]
          + jnp.dot(h_ref[...], root_ref[...],
                    preferred_element_type=jnp.float32, precision=_PREC)
          + bias_ref[...])
    if relu:
        hn = jnp.maximum(hn, 0.0)
    rows = lax.broadcasted_iota(jnp.int32, (_NP, _WN), 0) < _N
    out_ref[...] = jnp.where(rows, hn, 0.0)


def _final_body(h_ref, w2_ref, b2_ref, w3t_ref, b3_ref, out_ref):
    h2 = jnp.maximum(
        jnp.dot(h_ref[...], w2_ref[...], preferred_element_type=jnp.float32,
                precision=_PREC) + b2_ref[...], 0.0)
    out_ref[...] = (jnp.sum(h2 * w3t_ref[...], axis=1, keepdims=True)
                    + b3_ref[...])


def _mlp(ea, k1_w, k1_b, k2_w, k2_b):
    blk = 2000
    return pl.pallas_call(
        _mlp_body,
        grid=(_E // blk,),
        in_specs=[
            pl.BlockSpec((blk, 4), lambda i: (i, 0)),
            pl.BlockSpec((4, _WK // 2), lambda i: (0, 0)),
            pl.BlockSpec((_WK // 2,), lambda i: (0,)),
            pl.BlockSpec((_WK // 2, _WK), lambda i: (0, 0)),
            pl.BlockSpec((_WK,), lambda i: (0,)),
        ],
        out_specs=pl.BlockSpec((blk, _WK), lambda i: (i, 0)),
        out_shape=jax.ShapeDtypeStruct((_E, _WK), jnp.float32),
        name="tcmlp",
    )(ea, k1_w, k1_b, k2_w, k2_b)


def _main(hs, cs, k3r, b3r):
    return pl.pallas_call(
        _main_body,
        grid=(_EP // _MB,),
        in_specs=[
            pl.BlockSpec((_MB, _WN), lambda i: (i, 0)),
            pl.BlockSpec((_MB, _WK), lambda i: (i, 0)),
            pl.BlockSpec((_WK * _WN, _WN), lambda i: (0, 0)),
            pl.BlockSpec((_WN, _WN), lambda i: (0, 0)),
        ],
        out_specs=pl.BlockSpec((_MG, _WN), lambda i: (i, 0)),
        out_shape=jax.ShapeDtypeStruct((_G, _WN), jnp.float32),
        name="tcmain",
    )(hs, cs, k3r, b3r)


def _update(agg2, h, deginv, root, bias, relu):
    return pl.pallas_call(
        functools.partial(_update_body, relu=relu),
        in_specs=[
            pl.BlockSpec((2, _NP, _WN), lambda: (0, 0, 0)),
            pl.BlockSpec((_NP, _WN), lambda: (0, 0)),
            pl.BlockSpec((_NP, 1), lambda: (0, 0)),
            pl.BlockSpec((_WN, _WN), lambda: (0, 0)),
            pl.BlockSpec((_WN,), lambda: (0,)),
        ],
        out_specs=pl.BlockSpec((_NP, _WN), lambda: (0, 0)),
        out_shape=jax.ShapeDtypeStruct((_NP, _WN), jnp.float32),
        name="tcupdate",
    )(agg2, h, deginv, root, bias)


def _final(h, fc2_w, fc2_b, fc3_w, fc3_b):
    return pl.pallas_call(
        _final_body,
        in_specs=[
            pl.BlockSpec((_NP, _WN), lambda: (0, 0)),
            pl.BlockSpec((_WN, 128), lambda: (0, 0)),
            pl.BlockSpec((128,), lambda: (0,)),
            pl.BlockSpec((1, 128), lambda: (0, 0)),
            pl.BlockSpec((1,), lambda: (0,)),
        ],
        out_specs=pl.BlockSpec((_NP, 1), lambda: (0, 0)),
        out_shape=jax.ShapeDtypeStruct((_NP, 1), jnp.float32),
        name="tcfinal",
    )(h, fc2_w, fc2_b, fc3_w.T, fc3_b)


def kernel(x, edge_index, edge_attr, fc1_w, fc1_b, k1_w, k1_b, k2_w, k2_b,
           k3_w, k3_b, root, conv_bias, fc2_w, fc2_b, fc3_w, fc3_b):
    src = edge_index[0]
    dst = edge_index[1]

    key = (dst.astype(jnp.uint32) << 18) | jnp.arange(_E, dtype=jnp.uint32)
    key_s = jnp.sort(key)
    dst_s = (key_s >> 18).astype(jnp.int32)
    starts = jnp.searchsorted(
        dst_s, jnp.arange(_N + 1, dtype=jnp.int32)).astype(jnp.int32)
    deg = starts[1:] - starts[:-1]
    gcnt = (deg + 7) // 8
    goff = jnp.concatenate(
        [jnp.zeros((1,), jnp.int32), jnp.cumsum(gcnt).astype(jnp.int32)])
    pad1 = jnp.zeros((_NP - _N - 1,), jnp.int32)
    goff_p = jnp.concatenate([goff, pad1])
    starts_p = jnp.concatenate([starts, pad1])
    deg_p = jnp.concatenate([deg, jnp.zeros((_NP - _N,), jnp.int32)])
    deginv = 1.0 / jnp.clip(deg.astype(jnp.float32), 1.0)
    deginv_ext = jnp.concatenate(
        [deginv, jnp.ones((_NP - _N,), jnp.float32)])[:, None]

    cs = _mlp(edge_attr, k1_w, k1_b, k2_w, k2_b)
    build, gatherk, scatterk = _sc_kernels()
    idx, cs_pad, gnode = build(lax.bitcast_convert_type(key_s, jnp.int32),
                               src, cs, goff_p, starts_p, deg_p)
    idx3 = idx.reshape(_NW, _NCH, _CH)

    k3r = k3_w.reshape(_WK * _WN, _WN)
    b3r = k3_b.reshape(_WN, _WN)

    h0 = x @ fc1_w + fc1_b
    h = jnp.zeros((_NP, _WN), jnp.float32).at[:_N].set(h0)
    zeros_np = jnp.zeros((_NP, _WN), jnp.float32)

    for d in range(_DEPTH):
        hs = gatherk(h, idx3)
        aggp = _main(hs, cs_pad, k3r, b3r)
        agg2 = scatterk(aggp, gnode, zeros_np)
        h = _update(agg2, h, deginv_ext, root, conv_bias, d != _DEPTH - 1)

    out = _final(h, fc2_w, fc2_b, fc3_w, fc3_b)
    return out[:_N]
